# baseline (speedup 1.0000x reference)
import functools

import jax
import jax.numpy as jnp
from jax import lax
from jax.experimental import pallas as pl
from jax.experimental.pallas import tpu as pltpu
from jax.experimental.pallas import tpu_sc as plsc

D_MODEL = 64
SCALE = 8.0

_NUM_CORES = 2
_NUM_SUBCORES = 16
_NW = _NUM_CORES * _NUM_SUBCORES
_C = 512
_TN = 400


def _lookup_body(seq, x_t_hbm, tab_hbm, out_hbm,
                 idx_a, idx_b, rows_a, rows_b, trans,
                 gsem_a, gsem_b, isem_a, isem_b):
    wid = lax.axis_index("s") * _NUM_CORES + lax.axis_index("c")
    tok0 = wid * _C
    iota = lax.iota(jnp.int32, 16)

    def transpose_store(rows_c, s):
        @plsc.parallel_loop(0, _C // 16, unroll=4)
        def _(blk):
            t_vec = iota + blk * 16
            tcl = lax.shift_right_logical(blk, 3)
            ci0 = lax.bitwise_and(blk, 7) * 16
            for dd in range(D_MODEL):
                tr, ri = dd // 8, dd % 8
                d_vec = jnp.full((16,), dd, jnp.int32)
                v = plsc.load_gather(rows_c, [t_vec, d_vec]) * SCALE
                trans[tr, tcl, ri, pl.ds(ci0, 16)] = v

        pltpu.sync_copy(trans, out_hbm.at[s, :, pl.ds(wid * 4, 4)])

    pltpu.sync_copy(x_t_hbm.at[0, pl.ds(tok0, _C)], idx_a)
    pltpu.async_copy(tab_hbm.at[idx_a], rows_a, gsem_a)
    pltpu.async_copy(x_t_hbm.at[1, pl.ds(tok0, _C)], idx_b, isem_b)

    @pl.loop(0, seq, step=2)
    def pos_loop(g):
        bufs = (
            (idx_a, rows_a, gsem_a, isem_a, idx_b, rows_b, gsem_b, isem_b),
            (idx_b, rows_b, gsem_b, isem_b, idx_a, rows_a, gsem_a, isem_a),
        )
        for j, (idx_c, rows_c, gsem_c, isem_c,
                idx_o, rows_o, gsem_o, isem_o) in enumerate(bufs):
            s = g + j
            pltpu.make_async_copy(tab_hbm.at[idx_c], rows_c, gsem_c).wait()

            @pl.when(s + 2 < seq)
            def _():
                pltpu.async_copy(
                    x_t_hbm.at[s + 2, pl.ds(tok0, _C)], idx_c, isem_c)

            @pl.when(s + 1 < seq)
            def _():
                pltpu.make_async_copy(
                    x_t_hbm.at[s + 1, pl.ds(tok0, _C)], idx_o, isem_o).wait()
                pltpu.async_copy(tab_hbm.at[idx_o], rows_o, gsem_o)

            transpose_store(rows_c, s)


def kernel(x, table):
    s0, seq = x.shape
    vocab, d = table.shape
    assert d == D_MODEL and s0 == _NW * _C and seq % 2 == 0
    x_t = x.T
    mesh = plsc.VectorSubcoreMesh(
        core_axis_name="c", subcore_axis_name="s",
        num_cores=_NUM_CORES, num_subcores=_NUM_SUBCORES)
    params = pltpu.CompilerParams(use_tc_tiling_on_sc=False, needs_layout_passes=False,
        disable_bounds_checks=True)


    out_t = pl.kernel(
        functools.partial(_lookup_body, seq),
        out_type=jax.ShapeDtypeStruct((seq, d // 8, s0 // 128, 8, 128),
                                      jnp.float32),
        mesh=mesh,
        scratch_types=[
            pltpu.VMEM((_C,), jnp.int32),
            pltpu.VMEM((_C,), jnp.int32),
            pltpu.VMEM((_C, D_MODEL), jnp.float32),
            pltpu.VMEM((_C, D_MODEL), jnp.float32),
            pltpu.VMEM((D_MODEL // 8, _C // 128, 8, 128), jnp.float32),
            pltpu.SemaphoreType.DMA,
            pltpu.SemaphoreType.DMA,
            pltpu.SemaphoreType.DMA,
            pltpu.SemaphoreType.DMA,
        ],
        compiler_params=params,
    )(x_t, table)

    return out_t.transpose(2, 4, 0, 1, 3).reshape(s0, seq, d)


# speedup vs baseline: 1.5554x; 1.5554x over previous
import functools

import jax
import jax.numpy as jnp
from jax import lax
from jax.experimental import pallas as pl
from jax.experimental.pallas import tpu as pltpu
from jax.experimental.pallas import tpu_sc as plsc

D_MODEL = 64
SCALE = 8.0

_NUM_CORES = 2
_NUM_SUBCORES = 16
_NW = _NUM_CORES * _NUM_SUBCORES
_R = 4


def _emb_body(n_chunks, rows_per_w, seq, x_hbm, tab_hbm, out_hbm,
              idx_a, idx_b, rows_a, rows_b, gsem_a, gsem_b, isem_a, isem_b):
    wid = lax.axis_index("s") * _NUM_CORES + lax.axis_index("c")
    base = wid * rows_per_w

    def off(g):
        return base + g * _R

    def gather_start(idx_v, rows_v, sem):
        for r in range(_R):
            pltpu.async_copy(tab_hbm.at[idx_v.at[r]], rows_v.at[r], sem)

    def gather_wait(idx_v, rows_v, sem):
        for r in range(_R):
            pltpu.make_async_copy(
                tab_hbm.at[idx_v.at[r]], rows_v.at[r], sem).wait()

    def scale(rows):
        for r in range(_R):
            @pl.loop(0, seq, unroll=8)
            def _(i):
                for k in range(D_MODEL // 16):
                    sl = pl.ds(k * 16, 16)
                    rows[r, i, sl] = rows[r, i, sl] * SCALE

    pltpu.sync_copy(x_hbm.at[pl.ds(off(0), _R)], idx_a)
    gather_start(idx_a, rows_a, gsem_a)
    pltpu.async_copy(x_hbm.at[pl.ds(off(1), _R)], idx_b, isem_b)

    @pl.loop(0, n_chunks, step=2)
    def chunk_loop(g):
        bufs = (
            (idx_a, rows_a, gsem_a, isem_a, idx_b, rows_b, gsem_b, isem_b),
            (idx_b, rows_b, gsem_b, isem_b, idx_a, rows_a, gsem_a, isem_a),
        )
        for j, (idx_c, rows_c, gsem_c, isem_c,
                idx_o, rows_o, gsem_o, isem_o) in enumerate(bufs):
            cg = g + j
            gather_wait(idx_c, rows_c, gsem_c)

            @pl.when(cg + 2 < n_chunks)
            def _():
                pltpu.async_copy(
                    x_hbm.at[pl.ds(off(cg + 2), _R)], idx_c, isem_c)

            @pl.when(cg + 1 < n_chunks)
            def _():
                pltpu.make_async_copy(
                    x_hbm.at[pl.ds(off(cg + 1), _R)], idx_o, isem_o).wait()
                gather_start(idx_o, rows_o, gsem_o)

            scale(rows_c)
            pltpu.sync_copy(rows_c, out_hbm.at[pl.ds(off(cg), _R)])


def kernel(x, table):
    s0, seq = x.shape
    rows_per_w = s0 // _NW
    n_chunks = rows_per_w // _R
    assert n_chunks % 2 == 0
    mesh = plsc.VectorSubcoreMesh(
        core_axis_name="c", subcore_axis_name="s",
        num_cores=_NUM_CORES, num_subcores=_NUM_SUBCORES)
    out = pl.kernel(
        functools.partial(_emb_body, n_chunks, rows_per_w, seq),
        out_type=jax.ShapeDtypeStruct((s0, seq, D_MODEL), jnp.float32),
        mesh=mesh,
        scratch_types=[
            pltpu.VMEM((_R, seq), jnp.int32),
            pltpu.VMEM((_R, seq), jnp.int32),
            pltpu.VMEM((_R, seq, D_MODEL), jnp.float32),
            pltpu.VMEM((_R, seq, D_MODEL), jnp.float32),
            pltpu.SemaphoreType.DMA,
            pltpu.SemaphoreType.DMA,
            pltpu.SemaphoreType.DMA,
            pltpu.SemaphoreType.DMA,
        ],
        compiler_params=pltpu.CompilerParams(use_tc_tiling_on_sc=False),
    )(x, table)
    return out
